# baseline (speedup 1.0000x reference)
import functools

import jax
import jax.numpy as jnp
from jax import lax
from jax.experimental import pallas as pl
from jax.experimental.pallas import tpu as pltpu
from jax.experimental.pallas import tpu_sc as plsc

_VOCAB = 1000000
_D = 64
_BQ = 4096 * 20
_BD = 4096 * 200
_NC = 2
_NS = 16
_NW = _NC * _NS
_CH = 128
_QW = _BQ // _NW
_DW = _BD // _NW
_QC = _QW // _CH
_DC = _DW // _CH

_mesh = plsc.VectorSubcoreMesh(
    core_axis_name="c", subcore_axis_name="s", num_cores=_NC, num_subcores=_NS
)


@functools.partial(
    pl.kernel,
    out_type=(
        jax.ShapeDtypeStruct((_BQ, _D), jnp.float32),
        jax.ShapeDtypeStruct((_BD, _D), jnp.float32),
    ),
    mesh=_mesh,
    compiler_params=pltpu.CompilerParams(use_tc_tiling_on_sc=False),
    scratch_types=[
        pltpu.VMEM((_QW,), jnp.int32),
        pltpu.VMEM((_DW,), jnp.int32),
        pltpu.VMEM((_CH, _D), jnp.float32),
        pltpu.SemaphoreType.DMA,
    ],
)
def _embed_gather(q_hbm, d_hbm, table_hbm, qo_hbm, do_hbm, qidx, didx, rows, sem):
    w = lax.axis_index("s") * _NC + lax.axis_index("c")

    pltpu.sync_copy(q_hbm.at[pl.ds(w * _QW, _QW)], qidx)
    pltpu.sync_copy(d_hbm.at[pl.ds(w * _DW, _DW)], didx)

    def _shift(idx_ref, n):
        def body(i, carry):
            idx_ref[pl.ds(i * 16, 16)] = idx_ref[pl.ds(i * 16, 16)] + 1
            return carry
        lax.fori_loop(0, n // 16, body, 0)

    _shift(qidx, _QW)
    _shift(didx, _DW)

    def _gather(idx_ref, nchunks, out_hbm, base):
        def body(j, carry):
            idx = idx_ref.at[pl.ds(j * _CH, _CH)]
            pltpu.async_copy(table_hbm.at[idx], rows, sem).wait()
            pltpu.sync_copy(rows, out_hbm.at[pl.ds(base + j * _CH, _CH)])
            return carry
        lax.fori_loop(0, nchunks, body, 0)

    _gather(qidx, _QC, qo_hbm, w * _QW)
    _gather(didx, _DC, do_hbm, w * _DW)


def kernel(query_tok, doc_tok, table):
    q_idx = query_tok.reshape(_BQ).astype(jnp.int32)
    d_idx = doc_tok.reshape(_BD).astype(jnp.int32)
    q_emb, d_emb = _embed_gather(q_idx, d_idx, table)
    return (
        q_emb.reshape(*query_tok.shape, _D),
        d_emb.reshape(*doc_tok.shape, _D),
    )
